# baseline (speedup 1.0000x reference)
import functools

import jax
import jax.numpy as jnp
from jax import lax
from jax.experimental import pallas as pl
from jax.experimental.pallas import tpu as pltpu
from jax.experimental.pallas import tpu_sc as plsc

N = 10000
E = 320000
HID = 64
HEADS = 8
OUT_FEATS = 40

GA = 80
RB = 80
NSUB = 16
NBLK = N // RB


def _bcast_lane(v, lane):
    idx = jnp.full((16, 1), lane, jnp.int32)
    return lax.gather(
        v, idx,
        dimension_numbers=lax.GatherDimensionNumbers(
            offset_dims=(), collapsed_slice_dims=(0,), start_index_map=(0,)),
        slice_sizes=(1,),
        mode=lax.GatherScatterMode.PROMISE_IN_BOUNDS)


def _tc_mm_body(h_ref, w_ref, al_ref, ar_ref, z_ref, erl_ref,
                *, H, F, CZ, RW):
    C = h_ref.shape[0]
    bn = h_ref.shape[1]
    z = jnp.dot(h_ref[0], w_ref[0], preferred_element_type=jnp.float32)
    for c in range(1, C):
        z += jnp.dot(h_ref[c], w_ref[c], preferred_element_type=jnp.float32)
    zh = z.reshape(bn, H, F)
    el = jnp.sum(zh * al_ref[...][None], axis=-1)
    er = jnp.sum(zh * ar_ref[...][None], axis=-1)
    if H == 8:
        elr = jnp.concatenate([el, er], axis=1)
        erl = jnp.concatenate([er, el], axis=1)
    else:
        zpad = jnp.zeros((bn, 8 - H), jnp.float32)
        elr = jnp.concatenate([el, zpad, er, zpad], axis=1)
        erl = jnp.concatenate([er, zpad, el, zpad], axis=1)
    erl_ref[...] = erl
    HF = H * F
    if HF == CZ * RW:
        for c in range(CZ):
            z_ref[c] = jnp.concatenate([z[:, c * RW:(c + 1) * RW], elr],
                                       axis=1)
    else:
        z_ref[0] = jnp.concatenate([z[:, :RW], elr], axis=1)
        tail = HF - RW
        z_ref[1] = jnp.concatenate(
            [z[:, RW:], jnp.zeros((bn, RW - tail), jnp.float32), elr],
            axis=1)


def _tc_mm(hc, W, al, ar, *, H, F, CZ, RW):
    C, n, Kc = hc.shape
    HF = H * F
    TW = RW + 16
    Wr = W.reshape(C, Kc, HF)
    bn = 1000
    return pl.pallas_call(
        functools.partial(_tc_mm_body, H=H, F=F, CZ=CZ, RW=RW),
        grid=(n // bn,),
        in_specs=[
            pl.BlockSpec((C, bn, Kc), lambda i: (0, i, 0)),
            pl.BlockSpec((C, Kc, HF), lambda i: (0, 0, 0)),
            pl.BlockSpec(al.shape, lambda i: (0, 0)),
            pl.BlockSpec(ar.shape, lambda i: (0, 0)),
        ],
        out_specs=[
            pl.BlockSpec((CZ, bn, TW), lambda i: (0, i, 0)),
            pl.BlockSpec((bn, 16), lambda i: (i, 0)),
        ],
        out_shape=[
            jax.ShapeDtypeStruct((CZ, n, TW), jnp.float32),
            jax.ShapeDtypeStruct((n, 16), jnp.float32),
        ],
    )(hc, Wr, al, ar)


def _sc_edge(z_flat, erl, src, dst, *, H, CZ, RW, act):
    cps = CZ // 2
    QW = RW // 16
    TW = RW + 16
    NGT = E // GA // NSUB
    mesh = plsc.VectorSubcoreMesh(core_axis_name="c", subcore_axis_name="s")

    out_types = [jax.ShapeDtypeStruct((CZ * N, RW), jnp.float32)]
    if cps > 1:
        out_types.append(jax.ShapeDtypeStruct((2 * E, 16), jnp.float32))

    scratch = (
        [pltpu.VMEM((GA,), jnp.int32) for _ in range(6)]
        + [pltpu.VMEM((GA, 16), jnp.float32) for _ in range(4)]
        + [pltpu.VMEM((GA, TW), jnp.float32) for _ in range(2)]
        + [pltpu.VMEM((RB, RW), jnp.float32),
           pltpu.VMEM((RB, 16), jnp.float32),
           pltpu.VMEM_SHARED((N, RW), jnp.float32),
           pltpu.VMEM_SHARED((N, 16), jnp.float32)]
        + [pltpu.SemaphoreType.DMA for _ in range(4)]
    )

    @functools.partial(
        pl.kernel,
        mesh=mesh,
        compiler_params=pltpu.CompilerParams(use_tc_tiling_on_sc=False),
        out_type=out_types,
        scratch_types=scratch,
    )
    def k(src_h, dst_h, erl_h, z_h, *rest):
        if cps > 1:
            out_h, ex_h = rest[0], rest[1]
            rest = rest[2:]
        else:
            out_h = rest[0]
            ex_h = None
            rest = rest[1:]
        (sv0, sv1, dv0, dv1, sf0, sf1, rd0, rd1, xb0, xb1,
         zr0, zr1, ebuf, dbuf, acc, den, mz0, mz1, me0, me1) = rest
        srcv = (sv0, sv1)
        dstv = (dv0, dv1)
        srcf = (sf0, sf1)
        rd = (rd0, rd1)
        xb = (xb0, xb1)
        zr = (zr0, zr1)
        semz = (mz0, mz1)
        seme = (me0, me1)

        cid = lax.axis_index("c")
        sid = lax.axis_index("s")
        maskH = lax.iota(jnp.int32, 16) < H
        zv = jnp.zeros((16,), jnp.float32)
        gbase = sid * NGT
        exoff = cid * E

        nblk = jnp.where(sid < (NBLK % NSUB), NBLK // NSUB + 1, NBLK // NSUB)

        for p in range(cps):
            chunk = cid * cps + p
            off = chunk * N
            cached = p > 0

            def zrow(r, _):
                for q in range(QW):
                    ebuf[r, pl.ds(16 * q, 16)] = zv
                dbuf[r, pl.ds(0, 16)] = zv
                return 0
            lax.fori_loop(0, RB, zrow, 0)

            def zblk(i, _, p=p):
                r0 = (sid + i * NSUB) * RB
                pltpu.sync_copy(ebuf, acc.at[pl.ds(r0, RB)])
                if p == 0:
                    pltpu.sync_copy(dbuf, den.at[pl.ds(r0, RB)])
                return 0
            lax.fori_loop(0, nblk, zblk, 0)
            plsc.subcore_barrier()

            offv = jnp.full((16,), off, jnp.int32)

            def start(g, b, cached=cached):
                base = (gbase + g) * GA
                pltpu.sync_copy(src_h.at[pl.ds(base, GA)], srcv[b])
                pltpu.sync_copy(dst_h.at[pl.ds(base, GA)], dstv[b])
                for q in range(GA // 16):
                    srcf[b][pl.ds(16 * q, 16)] = (
                        srcv[b][pl.ds(16 * q, 16)] + offv)
                pltpu.async_copy(z_h.at[srcf[b]], zr[b], semz[b])
                if cached:
                    pltpu.async_copy(
                        ex_h.at[pl.ds(exoff + base, GA)], xb[b], seme[b])
                else:
                    pltpu.async_copy(erl_h.at[dstv[b]], rd[b], seme[b])

            def finish(g, b, p=p, cached=cached, chunk=chunk):
                base = (gbase + g) * GA
                pltpu.make_async_copy(z_h.at[srcf[b]], zr[b], semz[b]).wait()
                if cached:
                    pltpu.make_async_copy(
                        ex_h.at[pl.ds(exoff + base, GA)], xb[b],
                        seme[b]).wait()

                    def edge_b(j, _):
                        exv = xb[b][j, :]
                        if H == 8:
                            s0 = _bcast_lane(exv, 2 * chunk)
                            s1 = _bcast_lane(exv, 2 * chunk + 1)
                        else:
                            s0 = _bcast_lane(exv, 0)
                            s1 = s0
                        for q in range(QW):
                            sv = s0 if q < QW // 2 else s1
                            ebuf[j, pl.ds(16 * q, 16)] = (
                                zr[b][j, pl.ds(16 * q, 16)] * sv)
                        return 0
                    lax.fori_loop(0, GA, edge_b, 0, unroll=2)
                else:
                    pltpu.make_async_copy(
                        erl_h.at[dstv[b]], rd[b], seme[b]).wait()

                    def edge_ab(j, _):
                        s = zr[b][j, pl.ds(RW, 16)] + rd[b][j, :]
                        e = jnp.where(s > 0, s, 0.2 * s)
                        ex = jnp.exp(jnp.minimum(e, 60.0))
                        exm = jnp.where(maskH, ex, 0.0)
                        xb[b][j, :] = exm
                        if H == 8:
                            s0 = _bcast_lane(exm, 2 * chunk)
                            s1 = _bcast_lane(exm, 2 * chunk + 1)
                        else:
                            s0 = _bcast_lane(exm, 0)
                            s1 = s0
                        for q in range(QW):
                            sv = s0 if q < QW // 2 else s1
                            ebuf[j, pl.ds(16 * q, 16)] = (
                                zr[b][j, pl.ds(16 * q, 16)] * sv)
                        return 0
                    lax.fori_loop(0, GA, edge_ab, 0, unroll=2)

                    pltpu.sync_copy(xb[b], den.at[dstv[b]], add=True)
                    if cps > 1:
                        pltpu.sync_copy(
                            xb[b], ex_h.at[pl.ds(exoff + base, GA)])

                pltpu.sync_copy(ebuf, acc.at[dstv[b]], add=True)

            start(0, 0)

            def pair(i, _):
                g0 = 2 * i
                start(g0 + 1, 1)
                finish(g0, 0)

                @pl.when(g0 + 2 < NGT)
                def _():
                    start(g0 + 2, 0)
                finish(g0 + 1, 1)
                return 0
            lax.fori_loop(0, NGT // 2, pair, 0)
            plsc.subcore_barrier()

            def eblk(i, _, chunk=chunk, off=off):
                r0 = (sid + i * NSUB) * RB
                pltpu.sync_copy(acc.at[pl.ds(r0, RB)], ebuf)
                pltpu.sync_copy(den.at[pl.ds(r0, RB)], dbuf)

                def nrow(j, _):
                    dv = dbuf[j, :]
                    if H == 8:
                        d0 = _bcast_lane(dv, 2 * chunk) + 1e-9
                        d1 = _bcast_lane(dv, 2 * chunk + 1) + 1e-9
                    else:
                        d0 = _bcast_lane(dv, 0) + 1e-9
                        d1 = d0
                    for q in range(QW):
                        dq = d0 if q < QW // 2 else d1
                        v = ebuf[j, pl.ds(16 * q, 16)] / dq
                        if act:
                            v = jnp.where(v > 0, v, jnp.exp(v) - 1.0)
                        ebuf[j, pl.ds(16 * q, 16)] = v
                    return 0
                lax.fori_loop(0, RB, nrow, 0, unroll=2)
                pltpu.sync_copy(ebuf, out_h.at[pl.ds(off + r0, RB)])
                return 0
            lax.fori_loop(0, nblk, eblk, 0)
            plsc.subcore_barrier()

    res = k(src, dst, erl, z_flat)
    return res[0]


def kernel(x, edge_index, W1, al1, ar1, W2, al2, ar2, W3, al3, ar3):
    src = edge_index[0].astype(jnp.int32)
    dst = edge_index[1].astype(jnp.int32)

    z1, erl1 = _tc_mm(x.reshape(1, N, x.shape[1]), W1, al1, ar1,
                      H=HEADS, F=HID, CZ=4, RW=128)
    h1 = _sc_edge(z1.reshape(4 * N, 144), erl1, src, dst,
                  H=HEADS, CZ=4, RW=128, act=True)
    z2, erl2 = _tc_mm(h1.reshape(4, N, 128), W2, al2, ar2,
                      H=HEADS, F=HID, CZ=4, RW=128)
    h2 = _sc_edge(z2.reshape(4 * N, 144), erl2, src, dst,
                  H=HEADS, CZ=4, RW=128, act=True)
    z3, erl3 = _tc_mm(h2.reshape(4, N, 128), W3, al3, ar3,
                      H=1, F=OUT_FEATS, CZ=2, RW=32)
    o3 = _sc_edge(z3.reshape(2 * N, 48), erl3, src, dst,
                  H=1, CZ=2, RW=32, act=False)
    o3 = o3.reshape(2, N, 32)
    return jnp.concatenate([o3[0], o3[1]], axis=1)[:, :OUT_FEATS]


# speedup vs baseline: 1.8640x; 1.8640x over previous
import functools

import jax
import jax.numpy as jnp
from jax import lax
from jax.experimental import pallas as pl
from jax.experimental.pallas import tpu as pltpu
from jax.experimental.pallas import tpu_sc as plsc

N = 10000
E = 320000
HID = 64
HEADS = 8
OUT_FEATS = 40

def _bcast_lane(v, lane):
    idx = jnp.full((16, 1), lane, jnp.int32)
    return lax.gather(
        v, idx,
        dimension_numbers=lax.GatherDimensionNumbers(
            offset_dims=(), collapsed_slice_dims=(0,), start_index_map=(0,)),
        slice_sizes=(1,),
        mode=lax.GatherScatterMode.PROMISE_IN_BOUNDS)


GA = 80
NG = E // GA
RB = 80
NSUB = 16
NBLK = N // RB


def _tc_mm_body(h_ref, w_ref, al_ref, ar_ref, z_ref, elr_ref, erl_ref,
                *, H, F, CZ, RW):
    C = h_ref.shape[0]
    bn = h_ref.shape[1]
    z = jnp.dot(h_ref[0], w_ref[0], preferred_element_type=jnp.float32)
    for c in range(1, C):
        z += jnp.dot(h_ref[c], w_ref[c], preferred_element_type=jnp.float32)
    zh = z.reshape(bn, H, F)
    el = jnp.sum(zh * al_ref[...][None], axis=-1)
    er = jnp.sum(zh * ar_ref[...][None], axis=-1)
    if H == 8:
        elr_ref[...] = jnp.concatenate([el, er], axis=1)
        erl_ref[...] = jnp.concatenate([er, el], axis=1)
    else:
        zpad = jnp.zeros((bn, 8 - H), jnp.float32)
        elr_ref[...] = jnp.concatenate([el, zpad, er, zpad], axis=1)
        erl_ref[...] = jnp.concatenate([er, zpad, el, zpad], axis=1)
    HF = H * F
    if HF == CZ * RW:
        for c in range(CZ):
            z_ref[c] = z[:, c * RW:(c + 1) * RW]
    else:
        z_ref[0] = z[:, :RW]
        tail = HF - RW
        z_ref[1] = jnp.concatenate(
            [z[:, RW:], jnp.zeros((bn, RW - tail), jnp.float32)], axis=1)


def _tc_mm(hc, W, al, ar, *, H, F, CZ, RW):
    C, n, Kc = hc.shape
    HF = H * F
    Wr = W.reshape(C, Kc, HF)
    bn = 1000
    return pl.pallas_call(
        functools.partial(_tc_mm_body, H=H, F=F, CZ=CZ, RW=RW),
        grid=(n // bn,),
        in_specs=[
            pl.BlockSpec((C, bn, Kc), lambda i: (0, i, 0)),
            pl.BlockSpec((C, Kc, HF), lambda i: (0, 0, 0)),
            pl.BlockSpec(al.shape, lambda i: (0, 0)),
            pl.BlockSpec(ar.shape, lambda i: (0, 0)),
        ],
        out_specs=[
            pl.BlockSpec((CZ, bn, RW), lambda i: (0, i, 0)),
            pl.BlockSpec((bn, 16), lambda i: (i, 0)),
            pl.BlockSpec((bn, 16), lambda i: (i, 0)),
        ],
        out_shape=[
            jax.ShapeDtypeStruct((CZ, n, RW), jnp.float32),
            jax.ShapeDtypeStruct((n, 16), jnp.float32),
            jax.ShapeDtypeStruct((n, 16), jnp.float32),
        ],
    )(hc, Wr, al, ar)


def _sc_edge(z_flat, elr, erl, src, dst, *, H, CZ, RW, act):
    cps = CZ // 2
    QW = RW // 16
    NGT = E // GA // NSUB
    mesh = plsc.VectorSubcoreMesh(core_axis_name="c", subcore_axis_name="s")

    out_types = [jax.ShapeDtypeStruct((CZ * N, RW), jnp.float32)]
    if cps > 1:
        out_types.append(jax.ShapeDtypeStruct((2 * E, 16), jnp.float32))

    scratch = (
        [pltpu.VMEM((GA,), jnp.int32) for _ in range(6)]
        + [pltpu.VMEM((GA, 16), jnp.float32) for _ in range(6)]
        + [pltpu.VMEM((GA, RW), jnp.float32) for _ in range(2)]
        + [pltpu.VMEM((RB, RW), jnp.float32),
           pltpu.VMEM((RB, 16), jnp.float32),
           pltpu.VMEM_SHARED((N, RW), jnp.float32),
           pltpu.VMEM_SHARED((N, 16), jnp.float32)]
        + [pltpu.SemaphoreType.DMA for _ in range(12)]
    )

    @functools.partial(
        pl.kernel,
        mesh=mesh,
        compiler_params=pltpu.CompilerParams(use_tc_tiling_on_sc=False),
        out_type=out_types,
        scratch_types=scratch,
    )
    def k(src_h, dst_h, elr_h, erl_h, z_h, *rest):
        if cps > 1:
            out_h, ex_h = rest[0], rest[1]
            rest = rest[2:]
        else:
            out_h = rest[0]
            ex_h = None
            rest = rest[1:]
        (sv0, sv1, dv0, dv1, sf0, sf1, rs0, rs1, rd0, rd1, xb0, xb1,
         zr0, zr1, ebuf, dbuf, acc, den, mz0, mz1, me0, me1, mf0, mf1,
         mw0, mw1, md0, md1, mx0, mx1) = rest
        srcv = (sv0, sv1)
        dstv = (dv0, dv1)
        srcf = (sf0, sf1)
        rs = (rs0, rs1)
        rd = (rd0, rd1)
        xb = (xb0, xb1)
        zr = (zr0, zr1)
        semz = (mz0, mz1)
        seme = (me0, me1)
        semf = (mf0, mf1)
        semw = (mw0, mw1)
        semd = (md0, md1)
        semx = (mx0, mx1)

        cid = lax.axis_index("c")
        sid = lax.axis_index("s")
        maskH = lax.iota(jnp.int32, 16) < H
        zv = jnp.zeros((16,), jnp.float32)
        gbase = sid * NGT
        exoff = cid * E

        nblk = jnp.where(sid < (NBLK % NSUB), NBLK // NSUB + 1, NBLK // NSUB)

        for p in range(cps):
            chunk = cid * cps + p
            off = chunk * N
            cached = p > 0

            def zrow(r, _):
                for q in range(QW):
                    ebuf[r, pl.ds(16 * q, 16)] = zv
                dbuf[r, pl.ds(0, 16)] = zv
                return 0
            lax.fori_loop(0, RB, zrow, 0)

            def zblk(i, _, p=p):
                r0 = (sid + i * NSUB) * RB
                pltpu.sync_copy(ebuf, acc.at[pl.ds(r0, RB)])
                if p == 0:
                    pltpu.sync_copy(dbuf, den.at[pl.ds(r0, RB)])
                return 0
            lax.fori_loop(0, nblk, zblk, 0)
            plsc.subcore_barrier()

            offv = jnp.full((16,), off, jnp.int32)

            def start(g, b, p=p, cached=cached):
                base = (gbase + g) * GA
                pltpu.sync_copy(src_h.at[pl.ds(base, GA)], srcv[b])
                for q in range(GA // 16):
                    srcf[b][pl.ds(16 * q, 16)] = (
                        srcv[b][pl.ds(16 * q, 16)] + offv)
                if not cached:
                    pltpu.async_copy(elr_h.at[srcv[b]], rs[b], seme[b])

                @pl.when(g >= 2)
                def _():
                    pltpu.make_async_copy(zr[b], acc.at[dstv[b]],
                                          semw[b]).wait()
                    if p == 0:
                        pltpu.make_async_copy(xb[b], den.at[dstv[b]],
                                              semd[b]).wait()
                pltpu.async_copy(z_h.at[srcf[b]], zr[b], semz[b])
                pltpu.sync_copy(dst_h.at[pl.ds(base, GA)], dstv[b])
                if cached:
                    pltpu.async_copy(
                        ex_h.at[pl.ds(exoff + base, GA)], xb[b], seme[b])
                else:
                    pltpu.async_copy(erl_h.at[dstv[b]], rd[b], semf[b])

            def finish(g, b, p=p, cached=cached, chunk=chunk):
                base = (gbase + g) * GA
                if cached:
                    pltpu.make_async_copy(
                        ex_h.at[pl.ds(exoff + base, GA)], xb[b],
                        seme[b]).wait()
                else:
                    pltpu.make_async_copy(
                        elr_h.at[srcv[b]], rs[b], seme[b]).wait()
                    pltpu.make_async_copy(
                        erl_h.at[dstv[b]], rd[b], semf[b]).wait()
                    if cps > 1:
                        @pl.when(g >= 2)
                        def _():
                            pbase = (gbase + g - 2) * GA
                            pltpu.make_async_copy(
                                xb[b], ex_h.at[pl.ds(exoff + pbase, GA)],
                                semx[b]).wait()

                    def edge_a(j, _):
                        s = rs[b][j, :] + rd[b][j, :]
                        e = jnp.where(s > 0, s, 0.2 * s)
                        ex = jnp.exp(jnp.minimum(e, 60.0))
                        xb[b][j, :] = jnp.where(maskH, ex, 0.0)
                        return 0
                    lax.fori_loop(0, GA, edge_a, 0, unroll=2)

                    if p == 0:
                        pltpu.async_copy(xb[b], den.at[dstv[b]], semd[b],
                                         add=True)
                        if cps > 1:
                            pltpu.async_copy(
                                xb[b], ex_h.at[pl.ds(exoff + base, GA)],
                                semx[b])

                pltpu.make_async_copy(z_h.at[srcf[b]], zr[b], semz[b]).wait()

                def edge_b(j, _):
                    exv = xb[b][j, :]
                    if H == 8:
                        s0 = _bcast_lane(exv, 2 * chunk)
                        s1 = _bcast_lane(exv, 2 * chunk + 1)
                    else:
                        s0 = _bcast_lane(exv, 0)
                        s1 = s0
                    for q in range(QW):
                        sv = s0 if q < QW // 2 else s1
                        zr[b][j, pl.ds(16 * q, 16)] = (
                            zr[b][j, pl.ds(16 * q, 16)] * sv)
                    return 0
                lax.fori_loop(0, GA, edge_b, 0, unroll=2)

                pltpu.async_copy(zr[b], acc.at[dstv[b]], semw[b], add=True)

            start(0, 0)

            def pair(i, _):
                g0 = 2 * i
                start(g0 + 1, 1)
                finish(g0, 0)

                @pl.when(g0 + 2 < NGT)
                def _():
                    start(g0 + 2, 0)
                finish(g0 + 1, 1)
                return 0
            lax.fori_loop(0, NGT // 2, pair, 0)

            for tb, tg in ((0, NGT - 2), (1, NGT - 1)):
                pltpu.make_async_copy(zr[tb], acc.at[dstv[tb]],
                                      semw[tb]).wait()
                if p == 0:
                    pltpu.make_async_copy(xb[tb], den.at[dstv[tb]],
                                          semd[tb]).wait()
                    if cps > 1:
                        tbase = (gbase + tg) * GA
                        pltpu.make_async_copy(
                            xb[tb], ex_h.at[pl.ds(exoff + tbase, GA)],
                            semx[tb]).wait()
            plsc.subcore_barrier()

            def eblk(i, _, chunk=chunk, off=off):
                r0 = (sid + i * NSUB) * RB
                pltpu.sync_copy(acc.at[pl.ds(r0, RB)], ebuf)
                pltpu.sync_copy(den.at[pl.ds(r0, RB)], dbuf)

                def nrow(j, _):
                    dv = dbuf[j, :]
                    if H == 8:
                        d0 = _bcast_lane(dv, 2 * chunk) + 1e-9
                        d1 = _bcast_lane(dv, 2 * chunk + 1) + 1e-9
                    else:
                        d0 = _bcast_lane(dv, 0) + 1e-9
                        d1 = d0
                    for q in range(QW):
                        dq = d0 if q < QW // 2 else d1
                        v = ebuf[j, pl.ds(16 * q, 16)] / dq
                        if act:
                            v = jnp.where(v > 0, v, jnp.exp(v) - 1.0)
                        ebuf[j, pl.ds(16 * q, 16)] = v
                    return 0
                lax.fori_loop(0, RB, nrow, 0, unroll=2)
                pltpu.sync_copy(ebuf, out_h.at[pl.ds(off + r0, RB)])
                return 0
            lax.fori_loop(0, nblk, eblk, 0)
            plsc.subcore_barrier()

    res = k(src, dst, elr, erl, z_flat)
    return res[0]


def kernel(x, edge_index, W1, al1, ar1, W2, al2, ar2, W3, al3, ar3):
    src = edge_index[0].astype(jnp.int32)
    dst = edge_index[1].astype(jnp.int32)

    z1, elr1, erl1 = _tc_mm(x.reshape(1, N, x.shape[1]), W1, al1, ar1,
                            H=HEADS, F=HID, CZ=4, RW=128)
    h1 = _sc_edge(z1.reshape(4 * N, 128), elr1, erl1, src, dst,
                  H=HEADS, CZ=4, RW=128, act=True)
    z2, elr2, erl2 = _tc_mm(h1.reshape(4, N, 128), W2, al2, ar2,
                            H=HEADS, F=HID, CZ=4, RW=128)
    h2 = _sc_edge(z2.reshape(4 * N, 128), elr2, erl2, src, dst,
                  H=HEADS, CZ=4, RW=128, act=True)
    z3, elr3, erl3 = _tc_mm(h2.reshape(4, N, 128), W3, al3, ar3,
                            H=1, F=OUT_FEATS, CZ=2, RW=32)
    o3 = _sc_edge(z3.reshape(2 * N, 32), elr3, erl3, src, dst,
                  H=1, CZ=2, RW=32, act=False)
    o3 = o3.reshape(2, N, 32)
    return jnp.concatenate([o3[0], o3[1]], axis=1)[:, :OUT_FEATS]
